# Initial kernel scaffold; baseline (speedup 1.0000x reference)
import jax
import jax.numpy as jnp
from jax.experimental import pallas as pl


def kernel(emb_s, emb_t):
    raise NotImplementedError("write your pallas kernel here")


# speedup vs baseline: 72.9290x; 72.9290x over previous
import jax
import jax.numpy as jnp
from jax.experimental import pallas as pl

_N = 2048
_K = 32768
_ITERS = 45
_RB = 256


def _topk_mask_kernel(s_ref, t_ref, out_ref):
    s = s_ref[0, 0, :]
    t = t_ref[0, 0, :]
    n = s.shape[0]
    t_row = t[None, :]
    d = s * t

    smax, smin = jnp.max(s), jnp.min(s)
    tmax, tmin = jnp.max(t), jnp.min(t)
    hi0 = jnp.maximum(smax * tmax, smin * tmin)
    lo0 = jnp.minimum(smin * tmax, smax * tmin)

    def count_gt(m):
        cnt = jnp.float32(0.0)
        for rb in range(n // _RB):
            s_blk = s[rb * _RB:(rb + 1) * _RB][:, None]
            p = s_blk * t_row
            cnt = cnt + jnp.sum((p > m).astype(jnp.float32))
        cnt = cnt - jnp.sum((d > m).astype(jnp.float32))
        return cnt

    def body(_, carry):
        lo, hi = carry
        mid = 0.5 * (lo + hi)
        take = count_gt(mid) >= _K
        lo = jnp.where(take, mid, lo)
        hi = jnp.where(take, hi, mid)
        return lo, hi

    lo, hi = jax.lax.fori_loop(0, _ITERS, body, (lo0, hi0))

    for rb in range(n // _RB):
        r0 = rb * _RB
        s_blk = s[r0:r0 + _RB][:, None]
        p = s_blk * t_row
        rows = jax.lax.broadcasted_iota(jnp.int32, (_RB, n), 0) + r0
        cols = jax.lax.broadcasted_iota(jnp.int32, (_RB, n), 1)
        sel = (p > lo) & (rows != cols)
        out_ref[0, r0:r0 + _RB, :] = sel.astype(jnp.float32)


def kernel(emb_s, emb_t):
    g = emb_s.shape[0]
    s2 = emb_s.reshape(g, 1, _N)
    t2 = emb_t.reshape(g, 1, _N)
    return pl.pallas_call(
        _topk_mask_kernel,
        grid=(g,),
        in_specs=[
            pl.BlockSpec((1, 1, _N), lambda i: (i, 0, 0)),
            pl.BlockSpec((1, 1, _N), lambda i: (i, 0, 0)),
        ],
        out_specs=pl.BlockSpec((1, _N, _N), lambda i: (i, 0, 0)),
        out_shape=jax.ShapeDtypeStruct((g, _N, _N), jnp.float32),
    )(s2, t2)


# speedup vs baseline: 95.7491x; 1.3129x over previous
import jax
import jax.numpy as jnp
from jax.experimental import pallas as pl

_N = 2048
_K = 32768
_ITERS = 34
_RB = 256


def _topk_mask_kernel(s_ref, t_ref, out_ref):
    s = s_ref[0, 0, :]
    t = t_ref[0, 0, :]
    n = s.shape[0]
    t_row = t[None, :]
    d = s * t

    smax, smin = jnp.max(s), jnp.min(s)
    tmax, tmin = jnp.max(t), jnp.min(t)
    hi0 = jnp.maximum(smax * tmax, smin * tmin)
    lo0 = jnp.minimum(smin * tmax, smax * tmin)

    def count_gt(m):
        cnt = jnp.float32(0.0)
        for rb in range(n // _RB):
            s_blk = s[rb * _RB:(rb + 1) * _RB][:, None]
            p = s_blk * t_row
            cnt = cnt + jnp.sum((p > m).astype(jnp.float32))
        cnt = cnt - jnp.sum((d > m).astype(jnp.float32))
        return cnt

    def body(_, carry):
        lo, hi = carry
        mid = 0.5 * (lo + hi)
        take = count_gt(mid) >= _K
        lo = jnp.where(take, mid, lo)
        hi = jnp.where(take, hi, mid)
        return lo, hi

    lo, hi = jax.lax.fori_loop(0, _ITERS, body, (lo0, hi0))

    for rb in range(n // _RB):
        r0 = rb * _RB
        s_blk = s[r0:r0 + _RB][:, None]
        p = s_blk * t_row
        rows = jax.lax.broadcasted_iota(jnp.int32, (_RB, n), 0) + r0
        cols = jax.lax.broadcasted_iota(jnp.int32, (_RB, n), 1)
        sel = (p > lo) & (rows != cols)
        out_ref[0, r0:r0 + _RB, :] = sel.astype(jnp.float32)


def kernel(emb_s, emb_t):
    g = emb_s.shape[0]
    s2 = emb_s.reshape(g, 1, _N)
    t2 = emb_t.reshape(g, 1, _N)
    return pl.pallas_call(
        _topk_mask_kernel,
        grid=(g,),
        in_specs=[
            pl.BlockSpec((1, 1, _N), lambda i: (i, 0, 0)),
            pl.BlockSpec((1, 1, _N), lambda i: (i, 0, 0)),
        ],
        out_specs=pl.BlockSpec((1, _N, _N), lambda i: (i, 0, 0)),
        out_shape=jax.ShapeDtypeStruct((g, _N, _N), jnp.float32),
    )(s2, t2)
